# baseline (speedup 1.0000x reference)
import functools

import jax
import jax.numpy as jnp
from jax import lax
from jax.experimental import pallas as pl
from jax.experimental.pallas import tpu as pltpu
from jax.experimental.pallas import tpu_sc as plsc

VOCAB = 1_000_000
EMB = 64
ROWS = 4096 * 200
GRP = 128
NBUF = 8
LOOK = 4


def _make_gather():
    nc, ns = 2, 16
    nw = nc * ns
    rows_per_w = ROWS // nw
    ngrp = rows_per_w // GRP
    mesh = plsc.VectorSubcoreMesh(core_axis_name="c", subcore_axis_name="s")

    @functools.partial(
        pl.kernel,
        mesh=mesh,
        out_type=jax.ShapeDtypeStruct((ROWS, EMB), jnp.float32),
        scratch_types=[
            pltpu.VMEM((ngrp, GRP), jnp.int32),
            pltpu.VMEM((NBUF, GRP, EMB), jnp.float32),
        ] + [pltpu.SemaphoreType.DMA] * (2 * NBUF),
        compiler_params=pltpu.CompilerParams(use_tc_tiling_on_sc=False),
    )
    def gather_kernel(idx_hbm, table_hbm, out_hbm, idx_v, rows_v, *sems):
        sem_g = sems[:NBUF]
        sem_s = sems[NBUF:]
        wid = lax.axis_index("s") * nc + lax.axis_index("c")
        grp_base = wid * ngrp
        row_base = wid * rows_per_w

        pltpu.sync_copy(idx_hbm.at[pl.ds(grp_base, ngrp)], idx_v)

        def gather(g, b):
            pltpu.async_copy(table_hbm.at[idx_v.at[g]], rows_v.at[b],
                             sem_g[b])

        def wait_gather(g, b):
            pltpu.make_async_copy(table_hbm.at[idx_v.at[g]], rows_v.at[b],
                                  sem_g[b]).wait()

        def scatter(g, b):
            pltpu.async_copy(rows_v.at[b],
                             out_hbm.at[pl.ds(row_base + g * GRP, GRP)],
                             sem_s[b])

        def wait_scatter(g, b):
            pltpu.make_async_copy(rows_v.at[b],
                                  out_hbm.at[pl.ds(row_base + g * GRP, GRP)],
                                  sem_s[b]).wait()

        for g in range(LOOK):
            gather(g, g % NBUF)

        @pl.loop(0, ngrp, step=NBUF)
        def _(g0):
            for b in range(NBUF):
                g = g0 + b
                wait_gather(g, b)
                scatter(g, b)
                nxt = g + LOOK
                bn = (b + LOOK) % NBUF

                @pl.when(jnp.logical_and(nxt < ngrp, nxt >= NBUF))
                def _():
                    wait_scatter(nxt - NBUF, bn)
                    gather(nxt, bn)

                @pl.when(jnp.logical_and(nxt < ngrp, nxt < NBUF))
                def _():
                    gather(nxt, bn)

        for g in range(ngrp - NBUF, ngrp):
            wait_scatter(g, g % NBUF)

    return gather_kernel


_gather = _make_gather()


@jax.jit
def kernel(in_idx, token_emb):
    b, s = in_idx.shape
    idx2d = in_idx.astype(jnp.int32).reshape(ROWS // GRP, GRP)
    out = _gather(idx2d, token_emb)
    return out.reshape(b, s, EMB)


# speedup vs baseline: 1.0011x; 1.0011x over previous
import functools

import jax
import jax.numpy as jnp
from jax import lax
from jax.experimental import pallas as pl
from jax.experimental.pallas import tpu as pltpu
from jax.experimental.pallas import tpu_sc as plsc

VOCAB = 1_000_000
EMB = 64
BATCH = 4096
SEQ = 200
GRP = 200
NBUF = 4
LOOK = 2


def _make_gather():
    nc, ns = 2, 16
    nw = nc * ns
    b_per_w = BATCH // nw
    ngrp = b_per_w * SEQ // GRP
    mesh = plsc.VectorSubcoreMesh(core_axis_name="c", subcore_axis_name="s")

    @functools.partial(
        pl.kernel,
        mesh=mesh,
        out_type=jax.ShapeDtypeStruct((BATCH, SEQ, EMB), jnp.float32),
        scratch_types=[
            pltpu.VMEM((b_per_w, SEQ), jnp.int32),
            pltpu.VMEM((NBUF, GRP, EMB), jnp.float32),
        ] + [pltpu.SemaphoreType.DMA] * (2 * NBUF),
        compiler_params=pltpu.CompilerParams(use_tc_tiling_on_sc=False),
    )
    def gather_kernel(idx_hbm, table_hbm, out_hbm, idx_v, rows_v, *sems):
        sem_g = sems[:NBUF]
        sem_s = sems[NBUF:]
        wid = lax.axis_index("s") * nc + lax.axis_index("c")
        bbase = wid * b_per_w

        pltpu.sync_copy(idx_hbm.at[pl.ds(bbase, b_per_w)], idx_v)

        def idx_slice(g):
            return idx_v.at[g]

        def out_slice(g):
            return out_hbm.at[bbase + g]

        def gather(g, b):
            pltpu.async_copy(table_hbm.at[idx_slice(g)], rows_v.at[b],
                             sem_g[b])

        def wait_gather(g, b):
            pltpu.make_async_copy(table_hbm.at[idx_slice(g)], rows_v.at[b],
                                  sem_g[b]).wait()

        def scatter(g, b):
            pltpu.async_copy(rows_v.at[b], out_slice(g), sem_s[b])

        def wait_scatter(g, b):
            pltpu.make_async_copy(rows_v.at[b], out_slice(g), sem_s[b]).wait()

        for g in range(LOOK):
            gather(g, g % NBUF)

        @pl.loop(0, ngrp, step=NBUF)
        def _(g0):
            for b in range(NBUF):
                g = g0 + b
                wait_gather(g, b)
                scatter(g, b)
                nxt = g + LOOK
                bn = (b + LOOK) % NBUF

                @pl.when(jnp.logical_and(nxt < ngrp, nxt >= NBUF))
                def _():
                    wait_scatter(nxt - NBUF, bn)
                    gather(nxt, bn)

                @pl.when(jnp.logical_and(nxt < ngrp, nxt < NBUF))
                def _():
                    gather(nxt, bn)

        for g in range(ngrp - NBUF, ngrp):
            wait_scatter(g, g % NBUF)

    return gather_kernel


_gather = _make_gather()


@jax.jit
def kernel(in_idx, token_emb):
    return _gather(in_idx.astype(jnp.int32), token_emb)
